# baseline (speedup 1.0000x reference)
import jax
import jax.numpy as jnp
from jax.experimental import pallas as pl
from jax.experimental.pallas import tpu as pltpu

_N_TOK = 9216
_K = 8192
_D = 64
_BLK = 256
_NSTEPS = _N_TOK // _BLK
_COMMIT = 0.25


def _vq_kernel(x_ref, w_ref, enc_ref, q_ref, loss_ref, perp_ref,
               wnb_ref, wsq_ref, cnt_ref, wextb_ref, qe_ref):
    i = pl.program_id(0)

    @pl.when(i == 0)
    def _init():
        w = w_ref[...]
        n = jnp.sqrt(jnp.sum(w * w, axis=1, keepdims=True))
        wn = w / jnp.maximum(n, 1e-12)
        wnb_ref[...] = wn.astype(jnp.bfloat16)
        wsq_ref[...] = jnp.sum(wn * wn, axis=1)[None, :]
        wextb_ref[...] = jnp.concatenate(
            [w, jnp.ones((_K, 1), jnp.float32),
             jnp.zeros((_K, 128 - _D - 1), jnp.float32)],
            axis=1).astype(jnp.bfloat16)
        cnt_ref[...] = jnp.zeros_like(cnt_ref)
        loss_ref[...] = jnp.zeros_like(loss_ref)

    x = x_ref[...]
    xn_norm = jnp.sqrt(jnp.sum(x * x, axis=1, keepdims=True))
    x_n = x / jnp.maximum(xn_norm, 1e-12)

    s = jax.lax.dot_general(x_n.astype(jnp.bfloat16), wnb_ref[...],
                            (((1,), (1,)), ((), ())),
                            preferred_element_type=jnp.float32)
    xsq = jnp.sum(x_n * x_n, axis=1, keepdims=True)
    dist = (xsq + wsq_ref[...]) - 2.0 * s

    dmin = jnp.min(dist, axis=1, keepdims=True)
    mask = dist == dmin
    onehotf = mask.astype(jnp.float32)
    maskb = onehotf.astype(jnp.bfloat16)
    enc_ref[...] = onehotf
    cnt_ref[...] += jax.lax.dot_general(
        jnp.ones((1, _BLK), jnp.bfloat16), maskb, (((1,), (0,)), ((), ())),
        preferred_element_type=jnp.float32)

    qe = jax.lax.dot_general(maskb, wextb_ref[...],
                             (((1,), (0,)), ((), ())),
                             preferred_element_type=jnp.float32)
    qe_ref[...] = qe

    @pl.when(jnp.any(qe[:, _D] != 1.0))
    def _fix_ties():
        iota = jax.lax.broadcasted_iota(jnp.int32, (_BLK, _K), 1)
        idx = jnp.min(jnp.where(dist == dmin, iota, _K), axis=1)
        onehot = (iota == idx[:, None]).astype(jnp.float32)
        enc_ref[...] = onehot
        cnt_ref[...] += jnp.sum(onehot - mask.astype(jnp.float32),
                                axis=0, keepdims=True)
        qe_ref[...] = jax.lax.dot_general(
            onehot.astype(jnp.bfloat16), wextb_ref[...],
            (((1,), (0,)), ((), ())),
            preferred_element_type=jnp.float32)

    q = qe_ref[:, :_D]
    qn_norm = jnp.sqrt(jnp.sum(q * q, axis=1, keepdims=True))
    q_n = q / jnp.maximum(qn_norm, 1e-12)
    q_ref[...] = q_n

    diff = q_n - x_n
    loss_ref[...] += jnp.sum(diff * diff).reshape(1, 1)

    @pl.when(i == _NSTEPS - 1)
    def _fini():
        total = jnp.float32(_N_TOK * _D)
        loss_ref[...] = (1.0 + _COMMIT) * loss_ref[...] / total
        p = cnt_ref[...] / jnp.float32(_N_TOK)
        perp_ref[...] = jnp.exp(-jnp.sum(p * jnp.log(p + 1e-10))).reshape(1, 1)


@jax.jit
def kernel(f_emb, W):
    x = f_emb.reshape(-1, _D)

    grid = (_NSTEPS,)
    out = pl.pallas_call(
        _vq_kernel,
        grid=grid,
        in_specs=[
            pl.BlockSpec((_BLK, _D), lambda i: (i, 0)),
            pl.BlockSpec((_K, _D), lambda i: (0, 0)),
        ],
        out_specs=[
            pl.BlockSpec((_BLK, _K), lambda i: (i, 0)),
            pl.BlockSpec((_BLK, _D), lambda i: (i, 0)),
            pl.BlockSpec((1, 1), lambda i: (0, 0)),
            pl.BlockSpec((1, 1), lambda i: (0, 0)),
        ],
        out_shape=[
            jax.ShapeDtypeStruct((_N_TOK, _K), jnp.float32),
            jax.ShapeDtypeStruct((_N_TOK, _D), jnp.float32),
            jax.ShapeDtypeStruct((1, 1), jnp.float32),
            jax.ShapeDtypeStruct((1, 1), jnp.float32),
        ],
        scratch_shapes=[
            pltpu.VMEM((_K, _D), jnp.bfloat16),
            pltpu.VMEM((1, _K), jnp.float32),
            pltpu.VMEM((1, _K), jnp.float32),
            pltpu.VMEM((_K, 128), jnp.bfloat16),
            pltpu.VMEM((_BLK, 128), jnp.float32),
        ],
    )(x, W)

    encodings, quantized, loss, perp = out
    return (quantized.reshape(f_emb.shape), loss[0, 0], perp[0, 0], encodings)


# speedup vs baseline: 1.2705x; 1.2705x over previous
import jax
import jax.numpy as jnp
from jax.experimental import pallas as pl
from jax.experimental.pallas import tpu as pltpu

_N_TOK = 9216
_K = 8192
_D = 64
_BLK = 256
_NSTEPS = _N_TOK // _BLK
_COMMIT = 0.25


def _vq_kernel(x_ref, w_ref, enc_ref, q_ref, loss_ref, perp_ref,
               wnb_ref, wsq_ref, cnt_ref, wextb_ref, qe_ref):
    i = pl.program_id(0)

    @pl.when(i == 0)
    def _init():
        w = w_ref[...]
        n = jnp.sqrt(jnp.sum(w * w, axis=1, keepdims=True))
        wn = w / jnp.maximum(n, 1e-12)
        wnb_ref[...] = wn.astype(jnp.bfloat16)
        wsq_ref[...] = jnp.sum(wn * wn, axis=1)[None, :]
        wextb_ref[...] = jnp.concatenate(
            [w, jnp.ones((_K, 1), jnp.float32),
             jnp.zeros((_K, 128 - _D - 1), jnp.float32)],
            axis=1).astype(jnp.bfloat16)
        cnt_ref[...] = jnp.zeros_like(cnt_ref)
        loss_ref[...] = jnp.zeros_like(loss_ref)

    x = x_ref[...]
    xn_norm = jnp.sqrt(jnp.sum(x * x, axis=1, keepdims=True))
    x_n = x / jnp.maximum(xn_norm, 1e-12)

    s = jax.lax.dot_general(x_n.astype(jnp.bfloat16), wnb_ref[...],
                            (((1,), (1,)), ((), ())),
                            preferred_element_type=jnp.float32)
    xsq = jnp.sum(x_n * x_n, axis=1, keepdims=True)
    dist = (xsq + wsq_ref[...]) - 2.0 * s

    dmin = jnp.min(dist, axis=1, keepdims=True)
    mask = dist == dmin
    enc_ref[...] = mask.astype(jnp.float32)
    cnt_ref[...] += jnp.sum(mask.astype(jnp.float32), axis=0)[None, :]

    qe = jax.lax.dot_general(mask.astype(jnp.bfloat16), wextb_ref[...],
                             (((1,), (0,)), ((), ())),
                             preferred_element_type=jnp.float32)
    qe_ref[...] = qe

    @pl.when(jnp.any(qe[:, _D] != 1.0))
    def _fix_ties():
        iota = jax.lax.broadcasted_iota(jnp.int32, (_BLK, _K), 1)
        idx = jnp.min(jnp.where(dist == dmin, iota, _K), axis=1)
        onehot = (iota == idx[:, None]).astype(jnp.float32)
        enc_ref[...] = onehot
        cnt_ref[...] += jnp.sum(onehot - mask.astype(jnp.float32),
                                axis=0)[None, :]
        qe_ref[...] = jax.lax.dot_general(
            onehot.astype(jnp.bfloat16), wextb_ref[...],
            (((1,), (0,)), ((), ())),
            preferred_element_type=jnp.float32)

    q = qe_ref[:, :_D]
    qn_norm = jnp.sqrt(jnp.sum(q * q, axis=1, keepdims=True))
    q_n = q / jnp.maximum(qn_norm, 1e-12)
    q_ref[...] = q_n

    diff = q_n - x_n
    loss_ref[...] += jnp.sum(diff * diff).reshape(1, 1)

    @pl.when(i == _NSTEPS - 1)
    def _fini():
        total = jnp.float32(_N_TOK * _D)
        loss_ref[...] = (1.0 + _COMMIT) * loss_ref[...] / total
        p = cnt_ref[...] / jnp.float32(_N_TOK)
        perp_ref[...] = jnp.exp(-jnp.sum(p * jnp.log(p + 1e-10))).reshape(1, 1)


@jax.jit
def kernel(f_emb, W):
    x = f_emb.reshape(-1, _D)

    grid = (_NSTEPS,)
    out = pl.pallas_call(
        _vq_kernel,
        grid=grid,
        in_specs=[
            pl.BlockSpec((_BLK, _D), lambda i: (i, 0)),
            pl.BlockSpec((_K, _D), lambda i: (0, 0)),
        ],
        out_specs=[
            pl.BlockSpec((_BLK, _K), lambda i: (i, 0)),
            pl.BlockSpec((_BLK, _D), lambda i: (i, 0)),
            pl.BlockSpec((1, 1), lambda i: (0, 0)),
            pl.BlockSpec((1, 1), lambda i: (0, 0)),
        ],
        out_shape=[
            jax.ShapeDtypeStruct((_N_TOK, _K), jnp.float32),
            jax.ShapeDtypeStruct((_N_TOK, _D), jnp.float32),
            jax.ShapeDtypeStruct((1, 1), jnp.float32),
            jax.ShapeDtypeStruct((1, 1), jnp.float32),
        ],
        scratch_shapes=[
            pltpu.VMEM((_K, _D), jnp.bfloat16),
            pltpu.VMEM((1, _K), jnp.float32),
            pltpu.VMEM((1, _K), jnp.float32),
            pltpu.VMEM((_K, 128), jnp.bfloat16),
            pltpu.VMEM((_BLK, 128), jnp.float32),
        ],
    )(x, W)

    encodings, quantized, loss, perp = out
    return (quantized.reshape(f_emb.shape), loss[0, 0], perp[0, 0], encodings)
